# baseline (speedup 1.0000x reference)
import functools

import jax
import jax.numpy as jnp
from jax import lax
from jax.experimental import pallas as pl
from jax.experimental.pallas import tpu as pltpu
from jax.experimental.pallas import tpu_sc as plsc

N_ROWS = 16384
N_COLS = 832
N_FIELDS_K = 26
FIELD = 32
LANES = 16

NW = 32
ROWS_PER_W = N_ROWS // NW
CHUNK = 32
N_CHUNKS = ROWS_PER_W // CHUNK


def _sc_body(x_hbm, out_hbm, in_a, in_b, out_a, out_b,
             s_ia, s_ib, s_oa, s_ob):
    wid = lax.axis_index("s") * 2 + lax.axis_index("c")
    ji = lax.iota(jnp.int32, LANES)
    ji16 = ji + LANES
    row0 = wid * ROWS_PER_W

    def in_copy(k, buf, sem):
        return pltpu.make_async_copy(
            x_hbm.at[pl.ds(row0 + k * CHUNK, CHUNK)], buf, sem)

    def out_copy(k, buf, sem):
        return pltpu.make_async_copy(
            buf, out_hbm.at[pl.ds((row0 + k * CHUNK) * N_COLS,
                                  CHUNK * N_COLS)], sem)

    def compute(ibuf, obuf):
        def one_seg(r, c):
            v0 = ibuf[r, pl.ds(c, LANES)]
            v1 = ibuf[r, pl.ds(c + LANES, LANES)]
            m = jnp.max(jnp.maximum(v0, v1))
            f0 = plsc.all_reduce_ffs(v0 == m)
            f1 = plsc.all_reduce_ffs(v1 == m)
            first = jnp.where(f0 < LANES, f0, f1 + LANES)
            y0 = jnp.where(ji == first, 1.0, 0.0).astype(jnp.float32)
            y1 = jnp.where(ji16 == first, 1.0, 0.0).astype(jnp.float32)
            obuf[pl.ds(r * N_COLS + c, FIELD)] = plsc.pack(
                y0, y1, format=plsc.PackFormat.INTERLEAVED)

        @plsc.parallel_loop(0, CHUNK, unroll=2)
        def row_body(r):
            for f in range(N_FIELDS_K):
                one_seg(r, f * FIELD)

    in_copy(0, in_a, s_ia).start()
    in_copy(1, in_b, s_ib).start()

    in_copy(0, in_a, s_ia).wait()
    compute(in_a, out_a)
    out_copy(0, out_a, s_oa).start()
    in_copy(2, in_a, s_ia).start()

    in_copy(1, in_b, s_ib).wait()
    compute(in_b, out_b)
    out_copy(1, out_b, s_ob).start()
    in_copy(3, in_b, s_ib).start()

    def pair_body(jj, _):
        k0 = 2 * jj
        k1 = k0 + 1
        in_copy(k0, in_a, s_ia).wait()
        out_copy(k0 - 2, out_a, s_oa).wait()
        compute(in_a, out_a)
        out_copy(k0, out_a, s_oa).start()
        in_copy(k0 + 2, in_a, s_ia).start()

        in_copy(k1, in_b, s_ib).wait()
        out_copy(k1 - 2, out_b, s_ob).wait()
        compute(in_b, out_b)
        out_copy(k1, out_b, s_ob).start()
        in_copy(k1 + 2, in_b, s_ib).start()
        return 0

    lax.fori_loop(1, N_CHUNKS // 2 - 1, pair_body, 0)

    kl = N_CHUNKS - 2
    in_copy(kl, in_a, s_ia).wait()
    out_copy(kl - 2, out_a, s_oa).wait()
    compute(in_a, out_a)
    out_copy(kl, out_a, s_oa).start()

    in_copy(kl + 1, in_b, s_ib).wait()
    out_copy(kl - 1, out_b, s_ob).wait()
    compute(in_b, out_b)
    out_copy(kl + 1, out_b, s_ob).start()

    out_copy(kl, out_a, s_oa).wait()
    out_copy(kl + 1, out_b, s_ob).wait()


@jax.jit
def kernel(x):
    mesh = plsc.VectorSubcoreMesh(core_axis_name="c", subcore_axis_name="s")
    f = functools.partial(
        pl.kernel,
        mesh=mesh,
        out_type=jax.ShapeDtypeStruct((N_ROWS * N_COLS,), jnp.bfloat16),
        scratch_types=[
            pltpu.VMEM((CHUNK, N_COLS), jnp.float32),
            pltpu.VMEM((CHUNK, N_COLS), jnp.float32),
            pltpu.VMEM((CHUNK * N_COLS,), jnp.bfloat16),
            pltpu.VMEM((CHUNK * N_COLS,), jnp.bfloat16),
            pltpu.SemaphoreType.DMA,
            pltpu.SemaphoreType.DMA,
            pltpu.SemaphoreType.DMA,
            pltpu.SemaphoreType.DMA,
        ],
        compiler_params=pltpu.CompilerParams(needs_layout_passes=False),
    )(_sc_body)
    return f(x).reshape(N_ROWS, N_COLS).astype(jnp.float32)


# speedup vs baseline: 1.9726x; 1.9726x over previous
import functools

import jax
import jax.numpy as jnp
from jax import lax
from jax.experimental import pallas as pl
from jax.experimental.pallas import tpu as pltpu
from jax.experimental.pallas import tpu_sc as plsc

N_ROWS = 16384
N_COLS = 832
N_FIELDS_K = 26
FIELD = 32
LANES = 16

NW = 32
ROWS_PER_W = N_ROWS // NW
CHUNK = 32
N_CHUNKS = ROWS_PER_W // CHUNK


def _sc_body(x_hbm, out_hbm, in_a, in_b, out_a, out_b,
             s_ia, s_ib, s_oa, s_ob):
    wid = lax.axis_index("s") * 2 + lax.axis_index("c")
    ji = lax.iota(jnp.int32, LANES)
    ji16 = ji + LANES
    row0 = wid * ROWS_PER_W

    def in_copy(k, buf, sem):
        return pltpu.make_async_copy(
            x_hbm.at[pl.ds(row0 + k * CHUNK, CHUNK)], buf, sem)

    def out_copy(k, buf, sem):
        return pltpu.make_async_copy(
            buf, out_hbm.at[pl.ds(row0 + k * CHUNK, CHUNK)], sem)

    def compute(ibuf, obuf):
        def one_seg(r, c):
            v0 = ibuf[r, pl.ds(c, LANES)]
            v1 = ibuf[r, pl.ds(c + LANES, LANES)]
            m = jnp.max(jnp.maximum(v0, v1))
            f0 = plsc.all_reduce_ffs(v0 == m)
            f1 = plsc.all_reduce_ffs(v1 == m)
            first = jnp.where(f0 < LANES, f0, f1 + LANES)
            obuf[r, pl.ds(c, LANES)] = jnp.where(
                ji == first, 1.0, 0.0).astype(jnp.float32)
            obuf[r, pl.ds(c + LANES, LANES)] = jnp.where(
                ji16 == first, 1.0, 0.0).astype(jnp.float32)

        @plsc.parallel_loop(0, CHUNK, unroll=2)
        def row_body(r):
            for f in range(N_FIELDS_K):
                one_seg(r, f * FIELD)

    in_copy(0, in_a, s_ia).start()
    in_copy(1, in_b, s_ib).start()

    in_copy(0, in_a, s_ia).wait()
    compute(in_a, out_a)
    out_copy(0, out_a, s_oa).start()
    in_copy(2, in_a, s_ia).start()

    in_copy(1, in_b, s_ib).wait()
    compute(in_b, out_b)
    out_copy(1, out_b, s_ob).start()
    in_copy(3, in_b, s_ib).start()

    def pair_body(jj, _):
        k0 = 2 * jj
        k1 = k0 + 1
        in_copy(k0, in_a, s_ia).wait()
        out_copy(k0 - 2, out_a, s_oa).wait()
        compute(in_a, out_a)
        out_copy(k0, out_a, s_oa).start()
        in_copy(k0 + 2, in_a, s_ia).start()

        in_copy(k1, in_b, s_ib).wait()
        out_copy(k1 - 2, out_b, s_ob).wait()
        compute(in_b, out_b)
        out_copy(k1, out_b, s_ob).start()
        in_copy(k1 + 2, in_b, s_ib).start()
        return 0

    lax.fori_loop(1, N_CHUNKS // 2 - 1, pair_body, 0)

    kl = N_CHUNKS - 2
    in_copy(kl, in_a, s_ia).wait()
    out_copy(kl - 2, out_a, s_oa).wait()
    compute(in_a, out_a)
    out_copy(kl, out_a, s_oa).start()

    in_copy(kl + 1, in_b, s_ib).wait()
    out_copy(kl - 1, out_b, s_ob).wait()
    compute(in_b, out_b)
    out_copy(kl + 1, out_b, s_ob).start()

    out_copy(kl, out_a, s_oa).wait()
    out_copy(kl + 1, out_b, s_ob).wait()


@jax.jit
def kernel(x):
    mesh = plsc.VectorSubcoreMesh(core_axis_name="c", subcore_axis_name="s")
    f = functools.partial(
        pl.kernel,
        mesh=mesh,
        out_type=jax.ShapeDtypeStruct((N_ROWS, N_COLS), jnp.float32),
        scratch_types=[
            pltpu.VMEM((CHUNK, N_COLS), jnp.float32),
            pltpu.VMEM((CHUNK, N_COLS), jnp.float32),
            pltpu.VMEM((CHUNK, N_COLS), jnp.float32),
            pltpu.VMEM((CHUNK, N_COLS), jnp.float32),
            pltpu.SemaphoreType.DMA,
            pltpu.SemaphoreType.DMA,
            pltpu.SemaphoreType.DMA,
            pltpu.SemaphoreType.DMA,
        ],
        compiler_params=pltpu.CompilerParams(needs_layout_passes=False),
    )(_sc_body)
    return f(x)
